# baseline (speedup 1.0000x reference)
import functools

import jax
import jax.numpy as jnp
import numpy as np
from jax import lax
from jax.experimental import pallas as pl
from jax.experimental.pallas import tpu as pltpu
from jax.experimental.pallas import tpu_sc as plsc

_ALPHA = 4
_C, _T, _H, _W = 3, 64, 384, 384
_TS = _T // _ALPHA
_NROWS = _C * _T * _H
_OROWS = _C * _TS * _H
_NW = 32
_RPW = _OROWS // _NW
_CHUNK = 96
_STEPS = _RPW // _CHUNK


def _copy_body(src_ref, dst_ref):
    dst_ref[...] = src_ref[...]


def _gather_body(table_hbm, gidx_hbm, out_hbm, idx_v, rows_v, sem):
    nc = plsc.get_sparse_core_info().num_cores
    wid = lax.axis_index("s") * nc + lax.axis_index("c")
    pltpu.sync_copy(gidx_hbm.at[wid], idx_v)
    for s in range(_STEPS):
        pltpu.async_copy(table_hbm.at[idx_v.at[s]], rows_v, sem).wait()
        pltpu.sync_copy(rows_v, out_hbm.at[pl.ds(wid * _RPW + s * _CHUNK, _CHUNK)])


_IDX = np.linspace(0.0, _T - 1, _TS).astype(np.int32)
_G = (np.arange(_C, dtype=np.int32)[:, None] * _T + _IDX[None, :]).reshape(-1)
_GIDX = (_G[:, None] * _H
         + np.arange(_H, dtype=np.int32)[None, :]).reshape(_NW, _STEPS, _CHUNK)


@jax.jit
def _pack_pathway(frames):
    gidx = jnp.asarray(_GIDX)

    table = frames.reshape(_NROWS, _W)
    mesh = plsc.VectorSubcoreMesh(core_axis_name="c", subcore_axis_name="s")
    grab = functools.partial(
        pl.kernel,
        out_type=jax.ShapeDtypeStruct((_OROWS, _W), jnp.float32),
        mesh=mesh,
        scratch_types=[
            pltpu.VMEM((_STEPS, _CHUNK), jnp.int32),
            pltpu.VMEM((_CHUNK, _W), jnp.float32),
            pltpu.SemaphoreType.DMA,
        ],
    )(_gather_body)
    slow = grab(table, gidx).reshape(_C, _TS, _H, _W)

    blk = 8
    fast = pl.pallas_call(
        _copy_body,
        grid=(_C * _T // blk,),
        in_specs=[pl.BlockSpec((blk, _H, _W), lambda i: (i, 0, 0))],
        out_specs=pl.BlockSpec((blk, _H, _W), lambda i: (i, 0, 0)),
        out_shape=jax.ShapeDtypeStruct((_C * _T, _H, _W), jnp.float32),
    )(frames.reshape(_C * _T, _H, _W)).reshape(_C, _T, _H, _W)
    return slow, fast


def kernel(frames):
    return _pack_pathway(frames)


# speedup vs baseline: 1.0150x; 1.0150x over previous
import functools

import jax
import jax.numpy as jnp
import numpy as np
from jax import lax
from jax.experimental import pallas as pl
from jax.experimental.pallas import tpu as pltpu
from jax.experimental.pallas import tpu_sc as plsc

_ALPHA = 4
_C, _T, _H, _W = 3, 64, 384, 384
_TS = _T // _ALPHA
_NROWS = _C * _T * _H
_OROWS = _C * _TS * _H
_NW = 32
_RPW = _OROWS // _NW
_CHUNK = 96
_STEPS = _RPW // _CHUNK


def _copy_body(src_ref, dst_ref):
    dst_ref[...] = src_ref[...]


def _gather_body(table_hbm, gidx_hbm, out_hbm, idx_v, rows_v, sem):
    nc = plsc.get_sparse_core_info().num_cores
    wid = lax.axis_index("s") * nc + lax.axis_index("c")
    pltpu.sync_copy(gidx_hbm.at[wid], idx_v)

    @pl.loop(0, _STEPS)
    def _step(s):
        pltpu.async_copy(table_hbm.at[idx_v.at[s]], rows_v, sem).wait()
        pltpu.sync_copy(rows_v, out_hbm.at[pl.ds(wid * _RPW + s * _CHUNK, _CHUNK)])


_IDX = np.linspace(0.0, _T - 1, _TS).astype(np.int32)
_G = (np.arange(_C, dtype=np.int32)[:, None] * _T + _IDX[None, :]).reshape(-1)
_GIDX = (_G[:, None] * _H
         + np.arange(_H, dtype=np.int32)[None, :]).reshape(_NW, _STEPS, _CHUNK)


@jax.jit
def _pack_pathway(frames):
    gidx = jnp.asarray(_GIDX)

    blk = 16
    fast = pl.pallas_call(
        _copy_body,
        grid=(_C * _T // blk,),
        in_specs=[pl.BlockSpec((blk, _H, _W), lambda i: (i, 0, 0))],
        out_specs=pl.BlockSpec((blk, _H, _W), lambda i: (i, 0, 0)),
        out_shape=jax.ShapeDtypeStruct((_C * _T, _H, _W), jnp.float32),
    )(frames.reshape(_C * _T, _H, _W)).reshape(_C, _T, _H, _W)

    table = frames.reshape(_NROWS, _W)
    mesh = plsc.VectorSubcoreMesh(core_axis_name="c", subcore_axis_name="s")
    grab = functools.partial(
        pl.kernel,
        out_type=jax.ShapeDtypeStruct((_OROWS, _W), jnp.float32),
        mesh=mesh,
        scratch_types=[
            pltpu.VMEM((_STEPS, _CHUNK), jnp.int32),
            pltpu.VMEM((_CHUNK, _W), jnp.float32),
            pltpu.SemaphoreType.DMA,
        ],
    )(_gather_body)
    slow = grab(table, gidx).reshape(_C, _TS, _H, _W)
    return slow, fast


def kernel(frames):
    return _pack_pathway(frames)


# speedup vs baseline: 1.0226x; 1.0075x over previous
import functools

import jax
import jax.numpy as jnp
import numpy as np
from jax import lax
from jax.experimental import pallas as pl
from jax.experimental.pallas import tpu as pltpu
from jax.experimental.pallas import tpu_sc as plsc

_ALPHA = 4
_C, _T, _H, _W = 3, 64, 384, 384
_TS = _T // _ALPHA
_NROWS = _C * _T * _H
_OROWS = _C * _TS * _H
_NW = 32
_RPW = _OROWS // _NW
_CHUNK = 96
_STEPS = _RPW // _CHUNK


def _copy_body(src_ref, dst_ref):
    dst_ref[...] = src_ref[...]


def _gather_body(table_hbm, out_hbm, idx_v, rows_v, sem):
    nc = plsc.get_sparse_core_info().num_cores
    wid = lax.axis_index("s") * nc + lax.axis_index("c")

    @pl.loop(0, _STEPS)
    def _step(s):
        u = wid * _STEPS + s
        j = u // 4
        c, t = j // _TS, j % _TS
        src0 = (c * _T + (t * 63) // 15) * _H + (u % 4) * _CHUNK
        base = jnp.full((16,), src0, dtype=jnp.int32)
        lane = lax.iota(jnp.int32, 16)

        @pl.loop(0, _CHUNK // 16)
        def _fill(k):
            idx_v[pl.ds(k * 16, 16)] = base + k * 16 + lane

        pltpu.async_copy(table_hbm.at[idx_v], rows_v, sem).wait()
        pltpu.sync_copy(rows_v, out_hbm.at[pl.ds(wid * _RPW + s * _CHUNK, _CHUNK)])


_IDX = np.linspace(0.0, _T - 1, _TS).astype(np.int32)
_G = (np.arange(_C, dtype=np.int32)[:, None] * _T + _IDX[None, :]).reshape(-1)
_GIDX = (_G[:, None] * _H
         + np.arange(_H, dtype=np.int32)[None, :]).reshape(_NW, _STEPS, _CHUNK)


@jax.jit
def _pack_pathway(frames):
    blk = 16
    fast = pl.pallas_call(
        _copy_body,
        grid=(_C * _T // blk,),
        in_specs=[pl.BlockSpec((blk, _H, _W), lambda i: (i, 0, 0))],
        out_specs=pl.BlockSpec((blk, _H, _W), lambda i: (i, 0, 0)),
        out_shape=jax.ShapeDtypeStruct((_C * _T, _H, _W), jnp.float32),
    )(frames.reshape(_C * _T, _H, _W)).reshape(_C, _T, _H, _W)

    table = frames.reshape(_NROWS, _W)
    mesh = plsc.VectorSubcoreMesh(core_axis_name="c", subcore_axis_name="s")
    grab = functools.partial(
        pl.kernel,
        out_type=jax.ShapeDtypeStruct((_OROWS, _W), jnp.float32),
        mesh=mesh,
        scratch_types=[
            pltpu.VMEM((_CHUNK,), jnp.int32),
            pltpu.VMEM((_CHUNK, _W), jnp.float32),
            pltpu.SemaphoreType.DMA,
        ],
    )(_gather_body)
    slow = grab(table).reshape(_C, _TS, _H, _W)
    return slow, fast


def kernel(frames):
    return _pack_pathway(frames)


# speedup vs baseline: 1.0341x; 1.0113x over previous
import functools

import jax
import jax.numpy as jnp
import numpy as np
from jax import lax
from jax.experimental import pallas as pl
from jax.experimental.pallas import tpu as pltpu
from jax.experimental.pallas import tpu_sc as plsc

_ALPHA = 4
_C, _T, _H, _W = 3, 64, 384, 384
_TS = _T // _ALPHA
_NROWS = _C * _T * _H
_OROWS = _C * _TS * _H
_NW = 32
_RPW = _OROWS // _NW
_CHUNK = 96
_STEPS = _RPW // _CHUNK


def _copy_body(src_ref, dst_ref):
    dst_ref[...] = src_ref[...]


def _gather_body(table_hbm, out_hbm, idx_v, rows_v, sem):
    nc = plsc.get_sparse_core_info().num_cores
    wid = lax.axis_index("s") * nc + lax.axis_index("c")

    @pl.loop(0, _STEPS)
    def _step(s):
        u = wid * _STEPS + s
        j = u // 4
        c, t = j // _TS, j % _TS
        src0 = (c * _T + (t * 63) // 15) * _H + (u % 4) * _CHUNK
        base = jnp.full((16,), src0, dtype=jnp.int32)
        lane = lax.iota(jnp.int32, 16)

        @pl.loop(0, _CHUNK // 16)
        def _fill(k):
            idx_v[pl.ds(k * 16, 16)] = base + k * 16 + lane

        pltpu.async_copy(table_hbm.at[idx_v], rows_v, sem).wait()
        pltpu.sync_copy(rows_v, out_hbm.at[pl.ds(wid * _RPW + s * _CHUNK, _CHUNK)])


@jax.jit
def _pack_pathway(frames):
    blk = 24
    fast = pl.pallas_call(
        _copy_body,
        grid=(_C * _T // blk,),
        in_specs=[pl.BlockSpec((blk, _H, _W), lambda i: (i, 0, 0))],
        out_specs=pl.BlockSpec((blk, _H, _W), lambda i: (i, 0, 0)),
        out_shape=jax.ShapeDtypeStruct((_C * _T, _H, _W), jnp.float32),
    )(frames.reshape(_C * _T, _H, _W)).reshape(_C, _T, _H, _W)

    table = frames.reshape(_NROWS, _W)
    mesh = plsc.VectorSubcoreMesh(core_axis_name="c", subcore_axis_name="s")
    grab = functools.partial(
        pl.kernel,
        out_type=jax.ShapeDtypeStruct((_OROWS, _W), jnp.float32),
        mesh=mesh,
        scratch_types=[
            pltpu.VMEM((_CHUNK,), jnp.int32),
            pltpu.VMEM((_CHUNK, _W), jnp.float32),
            pltpu.SemaphoreType.DMA,
        ],
    )(_gather_body)
    slow = grab(table).reshape(_C, _TS, _H, _W)
    return slow, fast


def kernel(frames):
    return _pack_pathway(frames)
